# Initial kernel scaffold; baseline (speedup 1.0000x reference)
import jax
import jax.numpy as jnp
from jax.experimental import pallas as pl


def kernel(cls_score_0, cls_score_1, cls_score_2, bbox_pred_0, bbox_pred_1, bbox_pred_2, objectness_0, objectness_1, objectness_2):
    raise NotImplementedError("write your pallas kernel here")


# speedup vs baseline: 4.8703x; 4.8703x over previous
import functools

import jax
import jax.numpy as jnp
from jax import lax
from jax.experimental import pallas as pl
from jax.experimental.pallas import tpu as pltpu
from jax.experimental.pallas import tpu_sc as plsc

_NEG = -3.0e38
_TW = 128


def _sig(x):
    return 1.0 / (1.0 + jnp.exp(-x))


def _sc_partial_rows(cls0, cls1, cls2, bb0, bb1, bb2, ob0, ob1, ob2):
    info = plsc.get_sparse_core_info()
    nc, ns = info.num_cores, info.num_subcores
    nw = nc * ns

    mesh = plsc.VectorSubcoreMesh(core_axis_name="c", subcore_axis_name="s")

    scratch = [
        pltpu.VMEM((80, _TW), jnp.float32),
        pltpu.VMEM((4, _TW), jnp.float32),
        pltpu.VMEM((_TW,), jnp.float32),
        pltpu.VMEM((6, 16), jnp.float32),
        pltpu.SemaphoreType.DMA,
    ]

    @functools.partial(
        pl.kernel,
        mesh=mesh,
        out_type=jax.ShapeDtypeStruct((4 * nw, 6, 16), jnp.float32),
        scratch_types=scratch,
    )
    def body(c0h, c1h, c2h, b0h, b1h, b2h, o0h, o1h, o2h, out_hbm,
             cls_v, bb_v, ob_v, res_v, sem):
        wid = lax.axis_index("s") * nc + lax.axis_index("c")
        lane = lax.iota(jnp.int32, 16)

        def run_slot(slot, guard, clsh, bbh, obh, tile, gw, stride, base):
            @pl.when(guard)
            def _():
                a = tile * _TW
                cps = (
                    pltpu.async_copy(clsh.at[:, pl.ds(a, _TW)], cls_v, sem),
                    pltpu.async_copy(bbh.at[:, pl.ds(a, _TW)], bb_v, sem),
                    pltpu.async_copy(obh.at[pl.ds(a, _TW)], ob_v, sem),
                )
                for cp in cps:
                    cp.wait()

                bestv = jnp.full((16,), -1.0, jnp.float32)
                besti = jnp.full((16,), 2.0e9, jnp.float32)
                bx1 = jnp.zeros((16,), jnp.float32)
                by1 = jnp.zeros((16,), jnp.float32)
                bx2 = jnp.zeros((16,), jnp.float32)
                by2 = jnp.zeros((16,), jnp.float32)
                for g in range(_TW // 16):
                    sl = pl.ds(g * 16, 16)
                    c0 = cls_v[0, sl]
                    m = cls_v[1, sl]
                    for c in range(2, 80):
                        m = jnp.maximum(m, cls_v[c, sl])
                    person = c0 >= m
                    mall = jnp.maximum(c0, m)
                    key = jnp.where(person, _sig(ob_v[sl]) * _sig(mall),
                                    jnp.float32(-1.0))
                    il = a + g * 16 + lane
                    gx = (il % gw).astype(jnp.float32)
                    gy = (il.astype(jnp.float32) - gx) / gw
                    cx = (bb_v[0, sl] + gx) * stride
                    cy = (bb_v[1, sl] + gy) * stride
                    w_ = jnp.exp(bb_v[2, sl]) * stride
                    h_ = jnp.exp(bb_v[3, sl]) * stride
                    upd = key > bestv
                    bestv = jnp.where(upd, key, bestv)
                    besti = jnp.where(upd, (il + base).astype(jnp.float32),
                                      besti)
                    bx1 = jnp.where(upd, cx - w_ * 0.5, bx1)
                    by1 = jnp.where(upd, cy - h_ * 0.5, by1)
                    bx2 = jnp.where(upd, cx + w_ * 0.5, bx2)
                    by2 = jnp.where(upd, cy + h_ * 0.5, by2)

                res_v[0, :] = bestv
                res_v[1, :] = besti
                res_v[2, :] = bx1
                res_v[3, :] = by1
                res_v[4, :] = bx2
                res_v[5, :] = by2
                pltpu.sync_copy(res_v, out_hbm.at[slot * nw + wid])

            @pl.when(jnp.logical_not(guard))
            def _():
                res_v[0, :] = jnp.full((16,), -1.0, jnp.float32)
                res_v[1, :] = jnp.full((16,), 2.0e9, jnp.float32)
                res_v[2, :] = jnp.zeros((16,), jnp.float32)
                res_v[3, :] = jnp.zeros((16,), jnp.float32)
                res_v[4, :] = jnp.zeros((16,), jnp.float32)
                res_v[5, :] = jnp.zeros((16,), jnp.float32)
                pltpu.sync_copy(res_v, out_hbm.at[slot * nw + wid])

        true_ = wid >= 0
        run_slot(0, true_, c0h, b0h, o0h, wid, 80, 8.0, 0)
        run_slot(1, wid < 18, c0h, b0h, o0h, 32 + wid, 80, 8.0, 0)
        run_slot(2, wid >= 19, c1h, b1h, o1h, wid - 19, 40, 16.0, 6400)
        run_slot(3, (wid >= 14) & (wid < 18), c2h, b2h, o2h, wid - 14,
                 20, 32.0, 8000)

    return body(cls0, cls1, cls2, bb0, bb1, bb2, ob0, ob1, ob2)


def _merge_body(rows_ref, o_ref):
    rows = rows_ref[...]
    key = rows[:, 0, :]
    m = jnp.max(key)
    idxs = rows[:, 1, :]
    cand = key == m
    i_w = jnp.min(jnp.where(cand, idxs, jnp.float32(4.0e9)))
    m2 = cand & (idxs == i_w)
    x1 = jnp.max(jnp.where(m2, rows[:, 2, :], _NEG))
    y1 = jnp.max(jnp.where(m2, rows[:, 3, :], _NEG))
    x2 = jnp.max(jnp.where(m2, rows[:, 4, :], _NEG))
    y2 = jnp.max(jnp.where(m2, rows[:, 5, :], _NEG))
    use_real = m >= jnp.float32(0.47)
    ii = lax.broadcasted_iota(jnp.int32, (1, 5), 1)
    synth = jnp.where(ii >= 2, jnp.float32(640.0), jnp.float32(0.0))
    synth = jnp.where(ii == 4, jnp.float32(0.47), synth)
    real = jnp.where(ii == 0, x1, jnp.float32(0.0))
    real = jnp.where(ii == 1, y1, real)
    real = jnp.where(ii == 2, x2, real)
    real = jnp.where(ii == 3, y2, real)
    real = jnp.where(ii == 4, m, real)
    o_ref[...] = jnp.where(use_real, real, synth)


def _pad_level(cls, bb, ob, hw, hw_pad):
    pad = hw_pad - hw
    is_row0 = (jnp.arange(80, dtype=jnp.int32) == 0)[:, None]
    pad_cls = jnp.where(is_row0, jnp.float32(-1.0e6), jnp.float32(0.0))
    pad_cls = jnp.broadcast_to(pad_cls, (80, pad))
    return (jnp.concatenate([cls, pad_cls], axis=1),
            jnp.pad(bb, ((0, 0), (0, pad))),
            jnp.pad(ob, (0, pad)))


def kernel(cls_score_0, cls_score_1, cls_score_2, bbox_pred_0, bbox_pred_1,
           bbox_pred_2, objectness_0, objectness_1, objectness_2):
    cls1, bb1, ob1 = _pad_level(cls_score_1.reshape(80, 1600),
                                bbox_pred_1.reshape(4, 1600),
                                objectness_1.reshape(1600), 1600, 1664)
    cls2, bb2, ob2 = _pad_level(cls_score_2.reshape(80, 400),
                                bbox_pred_2.reshape(4, 400),
                                objectness_2.reshape(400), 400, 512)
    rows = _sc_partial_rows(
        cls_score_0.reshape(80, 6400), cls1, cls2,
        bbox_pred_0.reshape(4, 6400), bb1, bb2,
        objectness_0.reshape(6400), ob1, ob2,
    )
    return pl.pallas_call(
        _merge_body,
        out_shape=jax.ShapeDtypeStruct((1, 5), jnp.float32),
    )(rows)


# speedup vs baseline: 7.2165x; 1.4817x over previous
import functools

import jax
import jax.numpy as jnp
from jax import lax
from jax.experimental import pallas as pl
from jax.experimental.pallas import tpu as pltpu
from jax.experimental.pallas import tpu_sc as plsc

_NEG = -3.0e38


def _sig(x):
    return 1.0 / (1.0 + jnp.exp(-x))


def _sc_partial_rows(cls0, cls1, cls2, bb0, bb1, bb2, ob0, ob1, ob2):
    info = plsc.get_sparse_core_info()
    nc, ns = info.num_cores, info.num_subcores

    mesh = plsc.VectorSubcoreMesh(core_axis_name="c", subcore_axis_name="s")

    scratch = [
        pltpu.VMEM((80, 8, 80), jnp.float32),
        pltpu.VMEM((4, 8, 80), jnp.float32),
        pltpu.VMEM((8, 80), jnp.float32),
        pltpu.VMEM((80, 128), jnp.float32),
        pltpu.VMEM((4, 128), jnp.float32),
        pltpu.VMEM((128,), jnp.float32),
        pltpu.VMEM((6, 16), jnp.float32),
        pltpu.SemaphoreType.DMA,
    ]

    @functools.partial(
        pl.kernel,
        mesh=mesh,
        out_type=jax.ShapeDtypeStruct((27, 6, 16), jnp.float32),
        scratch_types=scratch,
    )
    def body(c0h, c1h, c2h, b0h, b1h, b2h, o0h, o1h, o2h, out_hbm,
             cls_v, bb_v, ob_v, clt_v, bbt_v, obt_v, res_v, sem):
        wid = lax.axis_index("s") * nc + lax.axis_index("c")
        lane = lax.iota(jnp.int32, 16)

        def init_res():
            res_v[0, :] = jnp.full((16,), -1.0, jnp.float32)
            res_v[1, :] = jnp.full((16,), 2.0e9, jnp.float32)
            res_v[2, :] = jnp.zeros((16,), jnp.float32)
            res_v[3, :] = jnp.zeros((16,), jnp.float32)
            res_v[4, :] = jnp.zeros((16,), jnp.float32)
            res_v[5, :] = jnp.zeros((16,), jnp.float32)

        def update(key, ilf, cx, cy, w_, h_):
            bvv = res_v[0, :]
            upd = key > bvv
            res_v[0, :] = jnp.where(upd, key, bvv)
            res_v[1, :] = jnp.where(upd, ilf, res_v[1, :])
            res_v[2, :] = jnp.where(upd, cx - w_ * 0.5, res_v[2, :])
            res_v[3, :] = jnp.where(upd, cy - h_ * 0.5, res_v[3, :])
            res_v[4, :] = jnp.where(upd, cx + w_ * 0.5, res_v[4, :])
            res_v[5, :] = jnp.where(upd, cy + h_ * 0.5, res_v[5, :])

        def keyed(c0v, m, obv):
            person = c0v >= m
            mall = jnp.maximum(c0v, m)
            return jnp.where(person, _sig(obv) * _sig(mall),
                             jnp.float32(-1.0))

        @pl.when(wid < 10)
        def _l0():
            h0 = wid * 8
            cps = (
                pltpu.async_copy(c0h.at[:, pl.ds(h0, 8), :], cls_v, sem),
                pltpu.async_copy(b0h.at[:, pl.ds(h0, 8), :], bb_v, sem),
                pltpu.async_copy(o0h.at[pl.ds(h0, 8), :], ob_v, sem),
            )
            for cp in cps:
                cp.wait()
            init_res()

            def row(hh, _):
                gy = (h0 + hh).astype(jnp.float32)
                rb = (h0 + hh) * 80
                for off in (0, 16, 32, 48, 64):
                    sl = pl.ds(off, 16)
                    c0v = cls_v[0, hh, sl]
                    m = cls_v[1, hh, sl]
                    for c in range(2, 80):
                        m = jnp.maximum(m, cls_v[c, hh, sl])
                    key = keyed(c0v, m, ob_v[hh, sl])
                    gx = (off + lane).astype(jnp.float32)
                    cx = (bb_v[0, hh, sl] + gx) * 8.0
                    cy = (bb_v[1, hh, sl] + gy) * 8.0
                    w_ = jnp.exp(bb_v[2, hh, sl]) * 8.0
                    h_ = jnp.exp(bb_v[3, hh, sl]) * 8.0
                    ilf = (rb + off + lane).astype(jnp.float32)
                    update(key, ilf, cx, cy, w_, h_)
                return _

            lax.fori_loop(0, 8, row, None)
            pltpu.sync_copy(res_v, out_hbm.at[wid])

        def tile_unit(unit, clsh, bbh, obh, tile, gw, stride, base):
            a = tile * 128
            cps = (
                pltpu.async_copy(clsh.at[:, pl.ds(a, 128)], clt_v, sem),
                pltpu.async_copy(bbh.at[:, pl.ds(a, 128)], bbt_v, sem),
                pltpu.async_copy(obh.at[pl.ds(a, 128)], obt_v, sem),
            )
            for cp in cps:
                cp.wait()
            init_res()

            def group(g, _):
                sl = pl.ds(g * 16, 16)
                c0v = clt_v[0, sl]
                m = clt_v[1, sl]
                for c in range(2, 80):
                    m = jnp.maximum(m, clt_v[c, sl])
                key = keyed(c0v, m, obt_v[sl])
                il = a + g * 16 + lane
                gx = (il % gw).astype(jnp.float32)
                gy = (il.astype(jnp.float32) - gx) / gw
                cx = (bbt_v[0, sl] + gx) * stride
                cy = (bbt_v[1, sl] + gy) * stride
                w_ = jnp.exp(bbt_v[2, sl]) * stride
                h_ = jnp.exp(bbt_v[3, sl]) * stride
                ilf = (il + base).astype(jnp.float32)
                update(key, ilf, cx, cy, w_, h_)
                return _

            lax.fori_loop(0, 8, group, None)
            pltpu.sync_copy(res_v, out_hbm.at[unit])

        @pl.when((wid >= 10) & (wid < 23))
        def _l1():
            tile_unit(wid, c1h, b1h, o1h, wid - 10, 40, 16.0, 6400)

        @pl.when((wid >= 23) & (wid < 27))
        def _l2():
            tile_unit(wid, c2h, b2h, o2h, wid - 23, 20, 32.0, 8000)

    return body(cls0, cls1, cls2, bb0, bb1, bb2, ob0, ob1, ob2)


def _merge_body(rows_ref, o_ref):
    rows = rows_ref[...]
    key = rows[:, 0, :]
    m = jnp.max(key)
    idxs = rows[:, 1, :]
    cand = key == m
    i_w = jnp.min(jnp.where(cand, idxs, jnp.float32(4.0e9)))
    m2 = cand & (idxs == i_w)
    x1 = jnp.max(jnp.where(m2, rows[:, 2, :], _NEG))
    y1 = jnp.max(jnp.where(m2, rows[:, 3, :], _NEG))
    x2 = jnp.max(jnp.where(m2, rows[:, 4, :], _NEG))
    y2 = jnp.max(jnp.where(m2, rows[:, 5, :], _NEG))
    use_real = m >= jnp.float32(0.47)
    ii = lax.broadcasted_iota(jnp.int32, (1, 5), 1)
    synth = jnp.where(ii >= 2, jnp.float32(640.0), jnp.float32(0.0))
    synth = jnp.where(ii == 4, jnp.float32(0.47), synth)
    real = jnp.where(ii == 0, x1, jnp.float32(0.0))
    real = jnp.where(ii == 1, y1, real)
    real = jnp.where(ii == 2, x2, real)
    real = jnp.where(ii == 3, y2, real)
    real = jnp.where(ii == 4, m, real)
    o_ref[...] = jnp.where(use_real, real, synth)


def _pad_level(cls, bb, ob, hw, hw_pad):
    pad = hw_pad - hw
    is_row0 = (jnp.arange(80, dtype=jnp.int32) == 0)[:, None]
    pad_cls = jnp.where(is_row0, jnp.float32(-1.0e6), jnp.float32(0.0))
    pad_cls = jnp.broadcast_to(pad_cls, (80, pad))
    return (jnp.concatenate([cls, pad_cls], axis=1),
            jnp.pad(bb, ((0, 0), (0, pad))),
            jnp.pad(ob, (0, pad)))


def kernel(cls_score_0, cls_score_1, cls_score_2, bbox_pred_0, bbox_pred_1,
           bbox_pred_2, objectness_0, objectness_1, objectness_2):
    cls1, bb1, ob1 = _pad_level(cls_score_1.reshape(80, 1600),
                                bbox_pred_1.reshape(4, 1600),
                                objectness_1.reshape(1600), 1600, 1664)
    cls2, bb2, ob2 = _pad_level(cls_score_2.reshape(80, 400),
                                bbox_pred_2.reshape(4, 400),
                                objectness_2.reshape(400), 400, 512)
    rows = _sc_partial_rows(
        cls_score_0.reshape(80, 80, 80), cls1, cls2,
        bbox_pred_0.reshape(4, 80, 80), bb1, bb2,
        objectness_0.reshape(80, 80), ob1, ob2,
    )
    return pl.pallas_call(
        _merge_body,
        out_shape=jax.ShapeDtypeStruct((1, 5), jnp.float32),
    )(rows)


# speedup vs baseline: 8.7649x; 1.2146x over previous
import functools

import jax
import jax.numpy as jnp
from jax import lax
from jax.experimental import pallas as pl
from jax.experimental.pallas import tpu as pltpu
from jax.experimental.pallas import tpu_sc as plsc

_NEG = -3.0e38


def _sig(x):
    return 1.0 / (1.0 + jnp.exp(-x))


def _sc_l0_rows(cls0, bb0, ob0):
    info = plsc.get_sparse_core_info()
    nc, ns = info.num_cores, info.num_subcores

    mesh = plsc.VectorSubcoreMesh(core_axis_name="c", subcore_axis_name="s")

    scratch = [
        pltpu.VMEM((80, 8, 80), jnp.float32),
        pltpu.VMEM((4, 8, 80), jnp.float32),
        pltpu.VMEM((8, 80), jnp.float32),
        pltpu.VMEM((6, 16), jnp.float32),
        pltpu.SemaphoreType.DMA,
    ]

    @functools.partial(
        pl.kernel,
        mesh=mesh,
        out_type=jax.ShapeDtypeStruct((10, 6, 16), jnp.float32),
        scratch_types=scratch,
    )
    def body(c0h, b0h, o0h, out_hbm, cls_v, bb_v, ob_v, res_v, sem):
        wid = lax.axis_index("s") * nc + lax.axis_index("c")
        lane = lax.iota(jnp.int32, 16)

        @pl.when(wid < 10)
        def _l0():
            h0 = wid * 8
            cps = (
                pltpu.async_copy(c0h.at[:, pl.ds(h0, 8), :], cls_v, sem),
                pltpu.async_copy(b0h.at[:, pl.ds(h0, 8), :], bb_v, sem),
                pltpu.async_copy(o0h.at[pl.ds(h0, 8), :], ob_v, sem),
            )
            for cp in cps:
                cp.wait()
            res_v[0, :] = jnp.full((16,), -1.0, jnp.float32)
            res_v[1, :] = jnp.full((16,), 2.0e9, jnp.float32)
            res_v[2, :] = jnp.zeros((16,), jnp.float32)
            res_v[3, :] = jnp.zeros((16,), jnp.float32)
            res_v[4, :] = jnp.zeros((16,), jnp.float32)
            res_v[5, :] = jnp.zeros((16,), jnp.float32)

            def row(hh, _):
                gy = (h0 + hh).astype(jnp.float32)
                rb = (h0 + hh) * 80
                for off in (0, 16, 32, 48, 64):
                    sl = pl.ds(off, 16)
                    c0v = cls_v[0, hh, sl]
                    m = cls_v[1, hh, sl]
                    for c in range(2, 80):
                        m = jnp.maximum(m, cls_v[c, hh, sl])
                    person = c0v >= m
                    mall = jnp.maximum(c0v, m)
                    key = jnp.where(person, _sig(ob_v[hh, sl]) * _sig(mall),
                                    jnp.float32(-1.0))
                    gx = (off + lane).astype(jnp.float32)
                    cx = (bb_v[0, hh, sl] + gx) * 8.0
                    cy = (bb_v[1, hh, sl] + gy) * 8.0
                    w_ = jnp.exp(bb_v[2, hh, sl]) * 8.0
                    h_ = jnp.exp(bb_v[3, hh, sl]) * 8.0
                    ilf = (rb + off + lane).astype(jnp.float32)
                    bvv = res_v[0, :]
                    upd = key > bvv
                    res_v[0, :] = jnp.where(upd, key, bvv)
                    res_v[1, :] = jnp.where(upd, ilf, res_v[1, :])
                    res_v[2, :] = jnp.where(upd, cx - w_ * 0.5, res_v[2, :])
                    res_v[3, :] = jnp.where(upd, cy - h_ * 0.5, res_v[3, :])
                    res_v[4, :] = jnp.where(upd, cx + w_ * 0.5, res_v[4, :])
                    res_v[5, :] = jnp.where(upd, cy + h_ * 0.5, res_v[5, :])
                return _

            lax.fori_loop(0, 8, row, None)
            pltpu.sync_copy(res_v, out_hbm.at[wid])

    return body(cls0, bb0, ob0)


def _tc_level(cls, bb, ob, gw, stride, base):
    c0 = cls[0]
    m = jnp.max(cls[1:], axis=0)
    person = c0 >= m
    key = jnp.where(person, _sig(ob) * _sig(jnp.maximum(c0, m)),
                    jnp.float32(-1.0))
    gy = lax.broadcasted_iota(jnp.int32, (gw, gw), 0).astype(jnp.float32)
    gx = lax.broadcasted_iota(jnp.int32, (gw, gw), 1).astype(jnp.float32)
    idx = gy * gw + gx + float(base)
    cx = (bb[0] + gx) * stride
    cy = (bb[1] + gy) * stride
    w_ = jnp.exp(bb[2]) * stride
    h_ = jnp.exp(bb[3]) * stride
    k = jnp.max(key)
    sel = key == k
    i_w = jnp.min(jnp.where(sel, idx, jnp.float32(4.0e9)))
    m2 = sel & (idx == i_w)
    x1 = jnp.max(jnp.where(m2, cx - w_ * 0.5, _NEG))
    y1 = jnp.max(jnp.where(m2, cy - h_ * 0.5, _NEG))
    x2 = jnp.max(jnp.where(m2, cx + w_ * 0.5, _NEG))
    y2 = jnp.max(jnp.where(m2, cy + h_ * 0.5, _NEG))
    return k, i_w, x1, y1, x2, y2


def _tc_l12_body(c1_ref, b1_ref, o1_ref, c2_ref, b2_ref, o2_ref, o_ref):
    k1, i1, a1, b1, c1, d1 = _tc_level(c1_ref[...], b1_ref[...], o1_ref[...],
                                       40, 16.0, 6400)
    k2, i2, a2, b2, c2, d2 = _tc_level(c2_ref[...], b2_ref[...], o2_ref[...],
                                       20, 32.0, 8000)
    take2 = k2 > k1
    vals = (jnp.where(take2, k2, k1), jnp.where(take2, i2, i1),
            jnp.where(take2, a2, a1), jnp.where(take2, b2, b1),
            jnp.where(take2, c2, c1), jnp.where(take2, d2, d1))
    ii = lax.broadcasted_iota(jnp.int32, (1, 8), 1)
    row = jnp.zeros((1, 8), jnp.float32)
    for j, v in enumerate(vals):
        row = jnp.where(ii == j, v, row)
    o_ref[...] = row


def _merge_body(rows_ref, tc_ref, o_ref):
    rows = rows_ref[...]
    tcr = tc_ref[...]
    key = rows[:, 0, :]
    m0 = jnp.max(key)
    idxs = rows[:, 1, :]
    cand = key == m0
    i0 = jnp.min(jnp.where(cand, idxs, jnp.float32(4.0e9)))
    m2 = cand & (idxs == i0)
    x1 = jnp.max(jnp.where(m2, rows[:, 2, :], _NEG))
    y1 = jnp.max(jnp.where(m2, rows[:, 3, :], _NEG))
    x2 = jnp.max(jnp.where(m2, rows[:, 4, :], _NEG))
    y2 = jnp.max(jnp.where(m2, rows[:, 5, :], _NEG))
    kt = tcr[0, 0]
    take_t = kt > m0
    m = jnp.where(take_t, kt, m0)
    x1 = jnp.where(take_t, tcr[0, 2], x1)
    y1 = jnp.where(take_t, tcr[0, 3], y1)
    x2 = jnp.where(take_t, tcr[0, 4], x2)
    y2 = jnp.where(take_t, tcr[0, 5], y2)
    use_real = m >= jnp.float32(0.47)
    ii = lax.broadcasted_iota(jnp.int32, (1, 5), 1)
    synth = jnp.where(ii >= 2, jnp.float32(640.0), jnp.float32(0.0))
    synth = jnp.where(ii == 4, jnp.float32(0.47), synth)
    real = jnp.where(ii == 0, x1, jnp.float32(0.0))
    real = jnp.where(ii == 1, y1, real)
    real = jnp.where(ii == 2, x2, real)
    real = jnp.where(ii == 3, y2, real)
    real = jnp.where(ii == 4, m, real)
    o_ref[...] = jnp.where(use_real, real, synth)


def kernel(cls_score_0, cls_score_1, cls_score_2, bbox_pred_0, bbox_pred_1,
           bbox_pred_2, objectness_0, objectness_1, objectness_2):
    rows = _sc_l0_rows(
        cls_score_0.reshape(80, 80, 80),
        bbox_pred_0.reshape(4, 80, 80),
        objectness_0.reshape(80, 80),
    )
    tcrow = pl.pallas_call(
        _tc_l12_body,
        out_shape=jax.ShapeDtypeStruct((1, 8), jnp.float32),
    )(
        cls_score_1.reshape(80, 40, 40),
        bbox_pred_1.reshape(4, 40, 40),
        objectness_1.reshape(40, 40),
        cls_score_2.reshape(80, 20, 20),
        bbox_pred_2.reshape(4, 20, 20),
        objectness_2.reshape(20, 20),
    )
    return pl.pallas_call(
        _merge_body,
        out_shape=jax.ShapeDtypeStruct((1, 5), jnp.float32),
    )(rows, tcrow)


# speedup vs baseline: 9.0986x; 1.0381x over previous
import functools

import jax
import jax.numpy as jnp
from jax import lax
from jax.experimental import pallas as pl
from jax.experimental.pallas import tpu as pltpu
from jax.experimental.pallas import tpu_sc as plsc

_NEG = -3.0e38


def _sig(x):
    return 1.0 / (1.0 + jnp.exp(-x))


def _sc_l0_rows(cls0, bb0, ob0):
    info = plsc.get_sparse_core_info()
    nc, ns = info.num_cores, info.num_subcores

    mesh = plsc.VectorSubcoreMesh(core_axis_name="c", subcore_axis_name="s")

    scratch = [
        pltpu.VMEM((80, 8, 80), jnp.float32),
        pltpu.VMEM((4, 8, 80), jnp.float32),
        pltpu.VMEM((8, 80), jnp.float32),
        pltpu.VMEM((6, 16), jnp.float32),
        pltpu.SemaphoreType.DMA,
    ]

    @functools.partial(
        pl.kernel,
        mesh=mesh,
        out_type=jax.ShapeDtypeStruct((20, 6, 16), jnp.float32),
        scratch_types=scratch,
    )
    def body(c0h, b0h, o0h, out_hbm, cls_v, bb_v, ob_v, res_v, sem):
        wid = lax.axis_index("s") * nc + lax.axis_index("c")
        lane = lax.iota(jnp.int32, 16)

        @pl.when(wid < 20)
        def _l0():
            blk = lax.shift_right_logical(wid, 1)
            h0 = blk * 8
            r0 = (wid & 1) * 4
            cps = (
                pltpu.async_copy(c0h.at[:, pl.ds(h0, 8), :], cls_v, sem),
                pltpu.async_copy(b0h.at[:, pl.ds(h0, 8), :], bb_v, sem),
                pltpu.async_copy(o0h.at[pl.ds(h0, 8), :], ob_v, sem),
            )
            for cp in cps:
                cp.wait()
            res_v[0, :] = jnp.full((16,), -1.0, jnp.float32)
            res_v[1, :] = jnp.full((16,), 2.0e9, jnp.float32)
            res_v[2, :] = jnp.zeros((16,), jnp.float32)
            res_v[3, :] = jnp.zeros((16,), jnp.float32)
            res_v[4, :] = jnp.zeros((16,), jnp.float32)
            res_v[5, :] = jnp.zeros((16,), jnp.float32)

            def row(hh, _):
                gy = (h0 + hh).astype(jnp.float32)
                rb = (h0 + hh) * 80
                for off in (0, 16, 32, 48, 64):
                    sl = pl.ds(off, 16)
                    c0v = cls_v[0, hh, sl]
                    m = cls_v[1, hh, sl]
                    for c in range(2, 80):
                        m = jnp.maximum(m, cls_v[c, hh, sl])
                    person = c0v >= m
                    mall = jnp.maximum(c0v, m)
                    key = jnp.where(person, _sig(ob_v[hh, sl]) * _sig(mall),
                                    jnp.float32(-1.0))
                    gx = (off + lane).astype(jnp.float32)
                    cx = (bb_v[0, hh, sl] + gx) * 8.0
                    cy = (bb_v[1, hh, sl] + gy) * 8.0
                    w_ = jnp.exp(bb_v[2, hh, sl]) * 8.0
                    h_ = jnp.exp(bb_v[3, hh, sl]) * 8.0
                    ilf = (rb + off + lane).astype(jnp.float32)
                    bvv = res_v[0, :]
                    upd = key > bvv
                    res_v[0, :] = jnp.where(upd, key, bvv)
                    res_v[1, :] = jnp.where(upd, ilf, res_v[1, :])
                    res_v[2, :] = jnp.where(upd, cx - w_ * 0.5, res_v[2, :])
                    res_v[3, :] = jnp.where(upd, cy - h_ * 0.5, res_v[3, :])
                    res_v[4, :] = jnp.where(upd, cx + w_ * 0.5, res_v[4, :])
                    res_v[5, :] = jnp.where(upd, cy + h_ * 0.5, res_v[5, :])
                return _

            lax.fori_loop(r0, r0 + 4, row, None)
            pltpu.sync_copy(res_v, out_hbm.at[wid])

    return body(cls0, bb0, ob0)


def _tc_level(cls, bb, ob, gw, stride, base):
    c0 = cls[0]
    m = jnp.max(cls[1:], axis=0)
    person = c0 >= m
    key = jnp.where(person, _sig(ob) * _sig(jnp.maximum(c0, m)),
                    jnp.float32(-1.0))
    gy = lax.broadcasted_iota(jnp.int32, (gw, gw), 0).astype(jnp.float32)
    gx = lax.broadcasted_iota(jnp.int32, (gw, gw), 1).astype(jnp.float32)
    idx = gy * gw + gx + float(base)
    cx = (bb[0] + gx) * stride
    cy = (bb[1] + gy) * stride
    w_ = jnp.exp(bb[2]) * stride
    h_ = jnp.exp(bb[3]) * stride
    k = jnp.max(key)
    sel = key == k
    i_w = jnp.min(jnp.where(sel, idx, jnp.float32(4.0e9)))
    m2 = sel & (idx == i_w)
    x1 = jnp.max(jnp.where(m2, cx - w_ * 0.5, _NEG))
    y1 = jnp.max(jnp.where(m2, cy - h_ * 0.5, _NEG))
    x2 = jnp.max(jnp.where(m2, cx + w_ * 0.5, _NEG))
    y2 = jnp.max(jnp.where(m2, cy + h_ * 0.5, _NEG))
    return k, i_w, x1, y1, x2, y2


def _tc_l12_body(c1_ref, b1_ref, o1_ref, c2_ref, b2_ref, o2_ref, o_ref):
    k1, i1, a1, b1, c1, d1 = _tc_level(c1_ref[...], b1_ref[...], o1_ref[...],
                                       40, 16.0, 6400)
    k2, i2, a2, b2, c2, d2 = _tc_level(c2_ref[...], b2_ref[...], o2_ref[...],
                                       20, 32.0, 8000)
    take2 = k2 > k1
    vals = (jnp.where(take2, k2, k1), jnp.where(take2, i2, i1),
            jnp.where(take2, a2, a1), jnp.where(take2, b2, b1),
            jnp.where(take2, c2, c1), jnp.where(take2, d2, d1))
    ii = lax.broadcasted_iota(jnp.int32, (1, 8), 1)
    row = jnp.zeros((1, 8), jnp.float32)
    for j, v in enumerate(vals):
        row = jnp.where(ii == j, v, row)
    o_ref[...] = row


def _merge_body(rows_ref, tc_ref, o_ref):
    rows = rows_ref[...]
    tcr = tc_ref[...]
    key = rows[:, 0, :]
    m0 = jnp.max(key)
    idxs = rows[:, 1, :]
    cand = key == m0
    i0 = jnp.min(jnp.where(cand, idxs, jnp.float32(4.0e9)))
    m2 = cand & (idxs == i0)
    x1 = jnp.max(jnp.where(m2, rows[:, 2, :], _NEG))
    y1 = jnp.max(jnp.where(m2, rows[:, 3, :], _NEG))
    x2 = jnp.max(jnp.where(m2, rows[:, 4, :], _NEG))
    y2 = jnp.max(jnp.where(m2, rows[:, 5, :], _NEG))
    kt = tcr[0, 0]
    take_t = kt > m0
    m = jnp.where(take_t, kt, m0)
    x1 = jnp.where(take_t, tcr[0, 2], x1)
    y1 = jnp.where(take_t, tcr[0, 3], y1)
    x2 = jnp.where(take_t, tcr[0, 4], x2)
    y2 = jnp.where(take_t, tcr[0, 5], y2)
    use_real = m >= jnp.float32(0.47)
    ii = lax.broadcasted_iota(jnp.int32, (1, 5), 1)
    synth = jnp.where(ii >= 2, jnp.float32(640.0), jnp.float32(0.0))
    synth = jnp.where(ii == 4, jnp.float32(0.47), synth)
    real = jnp.where(ii == 0, x1, jnp.float32(0.0))
    real = jnp.where(ii == 1, y1, real)
    real = jnp.where(ii == 2, x2, real)
    real = jnp.where(ii == 3, y2, real)
    real = jnp.where(ii == 4, m, real)
    o_ref[...] = jnp.where(use_real, real, synth)


def kernel(cls_score_0, cls_score_1, cls_score_2, bbox_pred_0, bbox_pred_1,
           bbox_pred_2, objectness_0, objectness_1, objectness_2):
    rows = _sc_l0_rows(
        cls_score_0.reshape(80, 80, 80),
        bbox_pred_0.reshape(4, 80, 80),
        objectness_0.reshape(80, 80),
    )
    tcrow = pl.pallas_call(
        _tc_l12_body,
        out_shape=jax.ShapeDtypeStruct((1, 8), jnp.float32),
    )(
        cls_score_1.reshape(80, 40, 40),
        bbox_pred_1.reshape(4, 40, 40),
        objectness_1.reshape(40, 40),
        cls_score_2.reshape(80, 20, 20),
        bbox_pred_2.reshape(4, 20, 20),
        objectness_2.reshape(20, 20),
    )
    return pl.pallas_call(
        _merge_body,
        out_shape=jax.ShapeDtypeStruct((1, 5), jnp.float32),
    )(rows, tcrow)
